# baseline (speedup 1.0000x reference)
import functools

import jax
import jax.numpy as jnp
from jax import lax
from jax.experimental import pallas as pl
from jax.experimental.pallas import tpu as pltpu
from jax.experimental.pallas import tpu_sc as plsc

EMBED = 32
LANES = 16
CHUNK = 128


def kernel(users, items, user_table, item_table):
    users = users.astype(jnp.int32)
    items = items.astype(jnp.int32)
    batch = users.shape[0]

    info = plsc.get_sparse_core_info()
    nc, ns = info.num_cores, info.num_subcores
    nw = nc * ns
    bpw = batch // nw
    nch = bpw // CHUNK

    mesh = plsc.VectorSubcoreMesh(core_axis_name="c", subcore_axis_name="s")

    @functools.partial(
        pl.kernel,
        mesh=mesh,
        compiler_params=pltpu.CompilerParams(
            needs_layout_passes=False, use_tc_tiling_on_sc=False),
        out_type=jax.ShapeDtypeStruct((batch,), jnp.float32),
        scratch_types=[
            pltpu.VMEM((nch, CHUNK), jnp.int32),
            pltpu.VMEM((nch, CHUNK), jnp.int32),
            pltpu.VMEM((bpw, EMBED), jnp.float32),
            pltpu.VMEM((bpw, EMBED), jnp.float32),
            pltpu.VMEM((bpw,), jnp.float32),
            pltpu.SemaphoreType.DMA,
        ],
    )
    def run(users_hbm, items_hbm, utab_hbm, itab_hbm, out_hbm,
            uidx, iidx, urows, irows, outv, sem):
        wid = lax.axis_index("s") * nc + lax.axis_index("c")
        base = wid * bpw

        for j in range(nch):
            pltpu.sync_copy(users_hbm.at[pl.ds(base + j * CHUNK, CHUNK)],
                            uidx.at[j])
            pltpu.sync_copy(items_hbm.at[pl.ds(base + j * CHUNK, CHUNK)],
                            iidx.at[j])

        copies = []
        for j in range(nch):
            copies.append(pltpu.async_copy(
                utab_hbm.at[uidx.at[j]],
                urows.at[pl.ds(j * CHUNK, CHUNK)], sem))
            copies.append(pltpu.async_copy(
                itab_hbm.at[iidx.at[j]],
                irows.at[pl.ds(j * CHUNK, CHUNK)], sem))
        for cp in copies:
            cp.wait()

        lane = lax.iota(jnp.int32, LANES)

        def group(g, carry):
            row0 = pl.multiple_of(g * LANES, LANES)
            rid = row0 + lane
            acc = jnp.zeros((LANES,), jnp.float32)
            for d in range(EMBED):
                cid = jnp.bitwise_and(lane + d, EMBED - 1)
                ug = plsc.load_gather(urows, [rid, cid])
                ig = plsc.load_gather(irows, [rid, cid])
                acc = acc + ug * ig
            outv[pl.ds(row0, LANES)] = acc
            return carry

        lax.fori_loop(0, bpw // LANES, group, 0)

        pltpu.sync_copy(outv, out_hbm.at[pl.ds(base, bpw)])

    return run(users, items, user_table, item_table)
